# baseline (speedup 1.0000x reference)
import jax
import jax.numpy as jnp
from jax.experimental import pallas as pl
from jax.experimental.pallas import tpu as pltpu

_CHUNKS = 8


def _chunk_copy(s_hbm, s_vmem, sem, g, buf, r, rows):
    return pltpu.make_async_copy(
        s_hbm.at[g, pl.ds(r * rows, rows), :],
        s_vmem.at[buf, pl.ds(r * rows, rows), :],
        sem.at[buf, r],
    )


def _gcn_kernel(x_ref, s_hbm, w1_ref, b1_ref, w2_ref, b2_ref, wp_ref,
                bp_ref, o_ref, s_vmem, sem):
    b = pl.program_id(0)
    nb = pl.num_programs(0)
    n = s_hbm.shape[1]
    rows = n // _CHUNKS
    buf = jax.lax.rem(b, 2)
    nxt = 1 - buf

    @pl.when(b == 0)
    def _():
        for r in range(_CHUNKS):
            _chunk_copy(s_hbm, s_vmem, sem, 0, 0, r, rows).start()

    @pl.when(b + 1 < nb)
    def _():
        for r in range(_CHUNKS):
            _chunk_copy(s_hbm, s_vmem, sem, b + 1, nxt, r, rows).start()

    t1 = jnp.dot(x_ref[0], w1_ref[...],
                 preferred_element_type=jnp.float32)
    t1b = t1.astype(jnp.bfloat16)

    sb_chunks = []
    t2_chunks = []
    for r in range(_CHUNKS):
        _chunk_copy(s_hbm, s_vmem, sem, b, buf, r, rows).wait()
        sc = s_vmem[buf, r * rows:(r + 1) * rows, :].astype(jnp.bfloat16)
        sb_chunks.append(sc)
        h1 = jnp.dot(sc, t1b, preferred_element_type=jnp.float32)
        h1 = jnp.maximum(h1 + b1_ref[...], 0.0)
        t2_chunks.append(jnp.dot(h1, w2_ref[...],
                                 preferred_element_type=jnp.float32))
    t2b = jnp.concatenate(t2_chunks, axis=0).astype(jnp.bfloat16)

    mx_parts = []
    sm_parts = []
    for r in range(_CHUNKS):
        h2 = jnp.dot(sb_chunks[r], t2b, preferred_element_type=jnp.float32)
        h2 = jnp.maximum(h2 + b2_ref[...], 0.0)
        mx_parts.append(jnp.max(h2, axis=0, keepdims=True))
        sm_parts.append(jnp.sum(h2, axis=0, keepdims=True))
    mx = jnp.max(jnp.concatenate(mx_parts, axis=0), axis=0, keepdims=True)
    sm = jnp.sum(jnp.concatenate(sm_parts, axis=0), axis=0, keepdims=True)

    cat = jnp.concatenate([mx, sm], axis=1)
    o_ref[0] = jnp.dot(cat, wp_ref[...],
                       preferred_element_type=jnp.float32) + bp_ref[...]


def kernel(x, support, W1, b1, W2, b2, Wp, bp):
    B, N, D_IN = x.shape
    H1 = W1.shape[1]
    H2 = W2.shape[1]
    OUT = Wp.shape[1]

    b1_2d = b1.reshape(1, H1)
    b2_2d = b2.reshape(1, H2)
    bp_2d = bp.reshape(1, OUT)

    out = pl.pallas_call(
        _gcn_kernel,
        grid=(B,),
        in_specs=[
            pl.BlockSpec((1, N, D_IN), lambda b: (b, 0, 0)),
            pl.BlockSpec(memory_space=pltpu.MemorySpace.HBM),
            pl.BlockSpec((D_IN, H1), lambda b: (0, 0)),
            pl.BlockSpec((1, H1), lambda b: (0, 0)),
            pl.BlockSpec((H1, H2), lambda b: (0, 0)),
            pl.BlockSpec((1, H2), lambda b: (0, 0)),
            pl.BlockSpec((2 * H2, OUT), lambda b: (0, 0)),
            pl.BlockSpec((1, OUT), lambda b: (0, 0)),
        ],
        out_specs=pl.BlockSpec((1, 1, OUT), lambda b: (b, 0, 0)),
        out_shape=jax.ShapeDtypeStruct((B, 1, OUT), jnp.float32),
        scratch_shapes=[
            pltpu.VMEM((2, N, N), jnp.float32),
            pltpu.SemaphoreType.DMA((2, _CHUNKS)),
        ],
        compiler_params=pltpu.CompilerParams(
            vmem_limit_bytes=100 * 1024 * 1024,
        ),
    )(x, support, W1, b1_2d, W2, b2_2d, Wp, bp_2d)
    return out.reshape(B, OUT)


# speedup vs baseline: 1.0013x; 1.0013x over previous
import jax
import jax.numpy as jnp
from jax.experimental import pallas as pl
from jax.experimental.pallas import tpu as pltpu

_CHUNKS = 8


def _start_copies(s_hbm, dst, sem, g, rows):
    for r in range(_CHUNKS):
        pltpu.make_async_copy(
            s_hbm.at[g, pl.ds(r * rows, rows), :],
            dst.at[pl.ds(r * rows, rows), :],
            sem.at[r],
        ).start()


def _compute(x_ref, s_hbm, s_buf, sem, w1_ref, b1_ref, w2_ref, b2_ref,
             wp_ref, bp_ref, o_ref, b, rows):
    t1 = jnp.dot(x_ref[0], w1_ref[...],
                 preferred_element_type=jnp.float32)
    t1b = t1.astype(jnp.bfloat16)

    sb_chunks = []
    t2_chunks = []
    for r in range(_CHUNKS):
        pltpu.make_async_copy(
            s_hbm.at[b, pl.ds(r * rows, rows), :],
            s_buf.at[pl.ds(r * rows, rows), :],
            sem.at[r],
        ).wait()
        sc = s_buf[r * rows:(r + 1) * rows, :].astype(jnp.bfloat16)
        sb_chunks.append(sc)
        h1 = jnp.dot(sc, t1b, preferred_element_type=jnp.float32)
        h1 = jnp.maximum(h1 + b1_ref[...], 0.0)
        t2_chunks.append(jnp.dot(h1, w2_ref[...],
                                 preferred_element_type=jnp.float32))
    t2b = jnp.concatenate(t2_chunks, axis=0).astype(jnp.bfloat16)

    mx_parts = []
    sm_parts = []
    for r in range(_CHUNKS):
        h2 = jnp.dot(sb_chunks[r], t2b, preferred_element_type=jnp.float32)
        h2 = jnp.maximum(h2 + b2_ref[...], 0.0)
        mx_parts.append(jnp.max(h2, axis=0, keepdims=True))
        sm_parts.append(jnp.sum(h2, axis=0, keepdims=True))
    mx = jnp.max(jnp.concatenate(mx_parts, axis=0), axis=0, keepdims=True)
    sm = jnp.sum(jnp.concatenate(sm_parts, axis=0), axis=0, keepdims=True)

    cat = jnp.concatenate([mx, sm], axis=1)
    o_ref[0] = jnp.dot(cat, wp_ref[...],
                       preferred_element_type=jnp.float32) + bp_ref[...]


def _gcn_kernel(x_ref, s_hbm, w1_ref, b1_ref, w2_ref, b2_ref, wp_ref,
                bp_ref, o_ref, s_a, s_b, sem_a, sem_b):
    b = pl.program_id(0)
    nb = pl.num_programs(0)
    n = s_hbm.shape[1]
    rows = n // _CHUNKS
    even = jax.lax.rem(b, 2) == 0

    @pl.when(b == 0)
    def _():
        _start_copies(s_hbm, s_a, sem_a, 0, rows)

    @pl.when((b + 1 < nb) & even)
    def _():
        _start_copies(s_hbm, s_b, sem_b, b + 1, rows)

    @pl.when((b + 1 < nb) & jnp.logical_not(even))
    def _():
        _start_copies(s_hbm, s_a, sem_a, b + 1, rows)

    @pl.when(even)
    def _():
        _compute(x_ref, s_hbm, s_a, sem_a, w1_ref, b1_ref, w2_ref,
                 b2_ref, wp_ref, bp_ref, o_ref, b, rows)

    @pl.when(jnp.logical_not(even))
    def _():
        _compute(x_ref, s_hbm, s_b, sem_b, w1_ref, b1_ref, w2_ref,
                 b2_ref, wp_ref, bp_ref, o_ref, b, rows)


def kernel(x, support, W1, b1, W2, b2, Wp, bp):
    B, N, D_IN = x.shape
    H1 = W1.shape[1]
    H2 = W2.shape[1]
    OUT = Wp.shape[1]

    b1_2d = b1.reshape(1, H1)
    b2_2d = b2.reshape(1, H2)
    bp_2d = bp.reshape(1, OUT)

    out = pl.pallas_call(
        _gcn_kernel,
        grid=(B,),
        in_specs=[
            pl.BlockSpec((1, N, D_IN), lambda b: (b, 0, 0)),
            pl.BlockSpec(memory_space=pltpu.MemorySpace.HBM),
            pl.BlockSpec((D_IN, H1), lambda b: (0, 0)),
            pl.BlockSpec((1, H1), lambda b: (0, 0)),
            pl.BlockSpec((H1, H2), lambda b: (0, 0)),
            pl.BlockSpec((1, H2), lambda b: (0, 0)),
            pl.BlockSpec((2 * H2, OUT), lambda b: (0, 0)),
            pl.BlockSpec((1, OUT), lambda b: (0, 0)),
        ],
        out_specs=pl.BlockSpec((1, 1, OUT), lambda b: (b, 0, 0)),
        out_shape=jax.ShapeDtypeStruct((B, 1, OUT), jnp.float32),
        scratch_shapes=[
            pltpu.VMEM((N, N), jnp.float32),
            pltpu.VMEM((N, N), jnp.float32),
            pltpu.SemaphoreType.DMA((_CHUNKS,)),
            pltpu.SemaphoreType.DMA((_CHUNKS,)),
        ],
        compiler_params=pltpu.CompilerParams(
            vmem_limit_bytes=100 * 1024 * 1024,
        ),
    )(x, support, W1, b1_2d, W2, b2_2d, Wp, bp_2d)
    return out.reshape(B, OUT)


# speedup vs baseline: 1.0917x; 1.0902x over previous
import jax
import jax.numpy as jnp
from jax.experimental import pallas as pl
from jax.experimental.pallas import tpu as pltpu

_CHUNKS_TOP = 4
_CHUNKS_BOT = 4


def _layer1_chunk(s_chunk, t1b, b1_ref, w2_ref):
    h1 = jnp.dot(s_chunk, t1b, preferred_element_type=jnp.float32)
    h1 = jnp.maximum(h1 + b1_ref[...], 0.0)
    return jnp.dot(h1, w2_ref[...], preferred_element_type=jnp.float32)


def _compute(x_ref, st_ref, s_bot, sem, w1_ref, b1_ref, w2_ref, b2_ref,
             wp_ref, bp_ref, o_ref, src_bot):
    half = st_ref.shape[1]
    rt = half // _CHUNKS_TOP
    rb = half // _CHUNKS_BOT

    t1 = jnp.dot(x_ref[0], w1_ref[...],
                 preferred_element_type=jnp.float32)
    t1b = t1.astype(jnp.bfloat16)

    sb_chunks = []
    t2_chunks = []
    for r in range(_CHUNKS_TOP):
        sc = st_ref[0, r * rt:(r + 1) * rt, :].astype(jnp.bfloat16)
        sb_chunks.append(sc)
        t2_chunks.append(_layer1_chunk(sc, t1b, b1_ref, w2_ref))
    for r in range(_CHUNKS_BOT):
        pltpu.make_async_copy(
            src_bot.at[pl.ds(half + r * rb, rb), :],
            s_bot.at[pl.ds(r * rb, rb), :],
            sem.at[r],
        ).wait()
        sc = s_bot[r * rb:(r + 1) * rb, :].astype(jnp.bfloat16)
        sb_chunks.append(sc)
        t2_chunks.append(_layer1_chunk(sc, t1b, b1_ref, w2_ref))
    t2b = jnp.concatenate(t2_chunks, axis=0).astype(jnp.bfloat16)

    mx_parts = []
    sm_parts = []
    for sc in sb_chunks:
        h2 = jnp.dot(sc, t2b, preferred_element_type=jnp.float32)
        h2 = jnp.maximum(h2 + b2_ref[...], 0.0)
        mx_parts.append(jnp.max(h2, axis=0, keepdims=True))
        sm_parts.append(jnp.sum(h2, axis=0, keepdims=True))
    mx = jnp.max(jnp.concatenate(mx_parts, axis=0), axis=0, keepdims=True)
    sm = jnp.sum(jnp.concatenate(sm_parts, axis=0), axis=0, keepdims=True)

    cat = jnp.concatenate([mx, sm], axis=1)
    o_ref[0] = jnp.dot(cat, wp_ref[...],
                       preferred_element_type=jnp.float32) + bp_ref[...]


def _start_copies(src, dst, sem, rb, half):
    for r in range(_CHUNKS_BOT):
        pltpu.make_async_copy(
            src.at[pl.ds(half + r * rb, rb), :],
            dst.at[pl.ds(r * rb, rb), :],
            sem.at[r],
        ).start()


def _gcn_kernel(x_ref, st_ref, sb_hbm, w1_ref, b1_ref, w2_ref, b2_ref,
                wp_ref, bp_ref, o_ref, s_a, s_b, sem_a, sem_b):
    b = pl.program_id(0)
    nb = pl.num_programs(0)
    half = st_ref.shape[1]
    rb = half // _CHUNKS_BOT
    even = jax.lax.rem(b, 2) == 0

    @pl.when(b == 0)
    def _():
        _start_copies(sb_hbm.at[0], s_a, sem_a, rb, half)

    @pl.when((b + 1 < nb) & even)
    def _():
        _start_copies(sb_hbm.at[b + 1], s_b, sem_b, rb, half)

    @pl.when((b + 1 < nb) & jnp.logical_not(even))
    def _():
        _start_copies(sb_hbm.at[b + 1], s_a, sem_a, rb, half)

    @pl.when(even)
    def _():
        _compute(x_ref, st_ref, s_a, sem_a, w1_ref, b1_ref, w2_ref,
                 b2_ref, wp_ref, bp_ref, o_ref, sb_hbm.at[b])

    @pl.when(jnp.logical_not(even))
    def _():
        _compute(x_ref, st_ref, s_b, sem_b, w1_ref, b1_ref, w2_ref,
                 b2_ref, wp_ref, bp_ref, o_ref, sb_hbm.at[b])


def kernel(x, support, W1, b1, W2, b2, Wp, bp):
    B, N, D_IN = x.shape
    H1 = W1.shape[1]
    H2 = W2.shape[1]
    OUT = Wp.shape[1]
    half = N // 2

    b1_2d = b1.reshape(1, H1)
    b2_2d = b2.reshape(1, H2)
    bp_2d = bp.reshape(1, OUT)

    out = pl.pallas_call(
        _gcn_kernel,
        grid=(B,),
        in_specs=[
            pl.BlockSpec((1, N, D_IN), lambda b: (b, 0, 0)),
            pl.BlockSpec((1, half, N), lambda b: (b, 0, 0)),
            pl.BlockSpec(memory_space=pltpu.MemorySpace.HBM),
            pl.BlockSpec((D_IN, H1), lambda b: (0, 0)),
            pl.BlockSpec((1, H1), lambda b: (0, 0)),
            pl.BlockSpec((H1, H2), lambda b: (0, 0)),
            pl.BlockSpec((1, H2), lambda b: (0, 0)),
            pl.BlockSpec((2 * H2, OUT), lambda b: (0, 0)),
            pl.BlockSpec((1, OUT), lambda b: (0, 0)),
        ],
        out_specs=pl.BlockSpec((1, 1, OUT), lambda b: (b, 0, 0)),
        out_shape=jax.ShapeDtypeStruct((B, 1, OUT), jnp.float32),
        scratch_shapes=[
            pltpu.VMEM((half, N), jnp.float32),
            pltpu.VMEM((half, N), jnp.float32),
            pltpu.SemaphoreType.DMA((_CHUNKS_BOT,)),
            pltpu.SemaphoreType.DMA((_CHUNKS_BOT,)),
        ],
        compiler_params=pltpu.CompilerParams(
            vmem_limit_bytes=100 * 1024 * 1024,
        ),
    )(x, support, support, W1, b1_2d, W2, b2_2d, Wp, bp_2d)
    return out.reshape(B, OUT)


# speedup vs baseline: 1.1895x; 1.0896x over previous
import jax
import jax.numpy as jnp
from jax.experimental import pallas as pl
from jax.experimental.pallas import tpu as pltpu

_CHUNKS = 8


def _gcn_kernel(x_ref, s_ref, w1_ref, b1_ref, w2_ref, b2_ref, wp_ref,
                bp_ref, o_ref):
    n = s_ref.shape[1]
    rows = n // _CHUNKS

    t1 = jnp.dot(x_ref[0], w1_ref[...],
                 preferred_element_type=jnp.float32)
    t1b = t1.astype(jnp.bfloat16)

    sb_chunks = []
    t2_chunks = []
    for r in range(_CHUNKS):
        sc = s_ref[0, r * rows:(r + 1) * rows, :].astype(jnp.bfloat16)
        sb_chunks.append(sc)
        h1 = jnp.dot(sc, t1b, preferred_element_type=jnp.float32)
        h1 = jnp.maximum(h1 + b1_ref[...], 0.0)
        t2_chunks.append(jnp.dot(h1, w2_ref[...],
                                 preferred_element_type=jnp.float32))
    t2b = jnp.concatenate(t2_chunks, axis=0).astype(jnp.bfloat16)

    mx_parts = []
    sm_parts = []
    for r in range(_CHUNKS):
        h2 = jnp.dot(sb_chunks[r], t2b, preferred_element_type=jnp.float32)
        h2 = jnp.maximum(h2 + b2_ref[...], 0.0)
        mx_parts.append(jnp.max(h2, axis=0, keepdims=True))
        sm_parts.append(jnp.sum(h2, axis=0, keepdims=True))
    mx = jnp.max(jnp.concatenate(mx_parts, axis=0), axis=0, keepdims=True)
    sm = jnp.sum(jnp.concatenate(sm_parts, axis=0), axis=0, keepdims=True)

    cat = jnp.concatenate([mx, sm], axis=1)
    o_ref[0] = jnp.dot(cat, wp_ref[...],
                       preferred_element_type=jnp.float32) + bp_ref[...]


def kernel(x, support, W1, b1, W2, b2, Wp, bp):
    B, N, D_IN = x.shape
    H1 = W1.shape[1]
    H2 = W2.shape[1]
    OUT = Wp.shape[1]

    b1_2d = b1.reshape(1, H1)
    b2_2d = b2.reshape(1, H2)
    bp_2d = bp.reshape(1, OUT)

    out = pl.pallas_call(
        _gcn_kernel,
        grid=(B,),
        in_specs=[
            pl.BlockSpec((1, N, D_IN), lambda b: (b, 0, 0)),
            pl.BlockSpec((1, N, N), lambda b: (b, 0, 0)),
            pl.BlockSpec((D_IN, H1), lambda b: (0, 0)),
            pl.BlockSpec((1, H1), lambda b: (0, 0)),
            pl.BlockSpec((H1, H2), lambda b: (0, 0)),
            pl.BlockSpec((1, H2), lambda b: (0, 0)),
            pl.BlockSpec((2 * H2, OUT), lambda b: (0, 0)),
            pl.BlockSpec((1, OUT), lambda b: (0, 0)),
        ],
        out_specs=pl.BlockSpec((1, 1, OUT), lambda b: (b, 0, 0)),
        out_shape=jax.ShapeDtypeStruct((B, 1, OUT), jnp.float32),
        compiler_params=pltpu.CompilerParams(
            vmem_limit_bytes=100 * 1024 * 1024,
            dimension_semantics=("parallel",),
        ),
    )(x, support, W1, b1_2d, W2, b2_2d, Wp, bp_2d)
    return out.reshape(B, OUT)
